# baseline (speedup 1.0000x reference)
import functools

import jax
import jax.numpy as jnp
from jax import lax
from jax.experimental import pallas as pl
from jax.experimental.pallas import tpu as pltpu
from jax.experimental.pallas import tpu_sc as plsc

B, T, FEAT, D, L, V, SUB = 8, 2048, 80, 256, 128, 8000, 4
EPS = 0.1
TE = T // SUB
N = B * L
TV = 1024
NV = (V + TV - 1) // TV
VP = NV * TV


def _make_sc_gather():
    info = plsc.get_sparse_core_info()
    nc, ns = info.num_cores, info.num_subcores
    nw = nc * ns
    b_per_w = N // nw
    mesh = plsc.VectorSubcoreMesh(core_axis_name="c", subcore_axis_name="s")

    @functools.partial(
        pl.kernel,
        mesh=mesh,
        out_type=jax.ShapeDtypeStruct((N, D), jnp.float32),
        scratch_types=[
            pltpu.VMEM((b_per_w,), jnp.int32),
            pltpu.VMEM((b_per_w, D), jnp.float32),
            pltpu.SemaphoreType.DMA,
        ],
    )
    def gather_k(idx_hbm, table_hbm, out_hbm, idx_v, rows_v, sem):
        wid = lax.axis_index("s") * nc + lax.axis_index("c")
        base = wid * b_per_w
        pltpu.sync_copy(idx_hbm.at[pl.ds(base, b_per_w)], idx_v)
        pltpu.async_copy(table_hbm.at[idx_v], rows_v, sem).wait()
        pltpu.sync_copy(rows_v, out_hbm.at[pl.ds(base, b_per_w)])

    return gather_k


_sc_gather_cache = []


def _sc_gather(ids, emb):
    if not _sc_gather_cache:
        _sc_gather_cache.append(_make_sc_gather())
    return _sc_gather_cache[0](ids, emb)


def _tc_body(x_ref, lens_ref, e_ref, lab_ref, pad_ref, wsp_ref, wenc_ref,
             benc_ref, wout_ref, out_ref, dec_ref, z_ref, sw_ref):
    i = pl.program_id(0)

    @pl.when(i == 0)
    def _encoder():
        h = jnp.dot(x_ref[...].astype(jnp.bfloat16),
                    wsp_ref[...].astype(jnp.bfloat16),
                    preferred_element_type=jnp.float32)
        h = jnp.dot(h.astype(jnp.bfloat16),
                    wenc_ref[...].astype(jnp.bfloat16),
                    preferred_element_type=jnp.float32)
        h = jnp.maximum(h + benc_ref[...], 0.0)
        enc_len = lens_ref[...] // SUB
        tmask = (lax.broadcasted_iota(jnp.int32, (B, TE), 1)
                 < enc_len).astype(jnp.float32)
        h = h.reshape(B, TE, D) * tmask[:, :, None]
        ctx = h.sum(axis=1) / jnp.maximum(enc_len, 1).astype(jnp.float32)
        dec_ref[...] = (e_ref[...].reshape(B, L, D)
                        + ctx[:, None, :]).reshape(N, D).astype(jnp.bfloat16)
        sw_ref[...] = jnp.zeros((D, 1), jnp.float32)

    wt = wout_ref[...]
    zb = jnp.dot(dec_ref[...], wt.astype(jnp.bfloat16),
                 preferred_element_type=jnp.float32)

    def _tail(zb, wt):
        valid = lax.broadcasted_iota(jnp.int32, (1, TV), 1) < (V - i * TV)
        return jnp.where(valid, zb, -30000.0), jnp.where(valid, wt, 0.0)

    zb, wt = lax.cond(i == NV - 1, _tail, lambda zb, wt: (zb, wt), zb, wt)
    sw_ref[...] += jnp.sum(wt, axis=1, keepdims=True)
    z_ref[:, pl.ds(i * TV, TV)] = zb.astype(jnp.bfloat16)

    @pl.when(i == NV - 1)
    def _finalize():
        def _mx(c, m):
            zc = z_ref[:, pl.ds(c * TV, TV)]
            return jnp.maximum(
                m, jnp.max(zc, axis=1, keepdims=True).astype(jnp.float32))

        m = lax.fori_loop(0, NV, _mx, jnp.full((N, 1), -30000.0, jnp.float32))

        def _acc(c, carry):
            s, zl = carry
            zc = z_ref[:, pl.ds(c * TV, TV)].astype(jnp.float32)
            s = s + jnp.sum(jnp.exp(zc - m), axis=1, keepdims=True)
            cols = c * TV + lax.broadcasted_iota(jnp.int32, (1, TV), 1)
            zl = zl + jnp.sum(jnp.where(cols == lab_ref[...], zc, 0.0),
                              axis=1, keepdims=True)
            return s, zl

        s, zl = lax.fori_loop(
            0, NV, _acc,
            (jnp.zeros((N, 1), jnp.float32), jnp.zeros((N, 1), jnp.float32)))
        lse = m + jnp.log(s)
        sz = jnp.dot(dec_ref[...].astype(jnp.float32), sw_ref[...],
                     preferred_element_type=jnp.float32)
        loss_tok = lse - (1.0 - EPS) * zl - EPS * (sz / V)
        tmask = 1.0 - pad_ref[...]
        loss = jnp.sum(loss_tok * tmask) / jnp.maximum(jnp.sum(tmask), 1.0)
        out_ref[...] = loss.reshape(1, 1)


def _tc_main(x2, lens2, e, lab2, pad2, w_sp, w_enc, b_enc2, w_out,
             interpret=False):
    return pl.pallas_call(
        _tc_body,
        grid=(NV,),
        in_specs=[
            pl.BlockSpec((B * TE, SUB * FEAT), lambda i: (0, 0)),
            pl.BlockSpec((B, 1), lambda i: (0, 0)),
            pl.BlockSpec((N, D), lambda i: (0, 0)),
            pl.BlockSpec((N, 1), lambda i: (0, 0)),
            pl.BlockSpec((N, 1), lambda i: (0, 0)),
            pl.BlockSpec((SUB * FEAT, D), lambda i: (0, 0)),
            pl.BlockSpec((D, D), lambda i: (0, 0)),
            pl.BlockSpec((1, D), lambda i: (0, 0)),
            pl.BlockSpec((D, TV), lambda i: (0, i)),
        ],
        out_specs=pl.BlockSpec((1, 1), lambda i: (0, 0)),
        out_shape=jax.ShapeDtypeStruct((1, 1), jnp.float32),
        scratch_shapes=[
            pltpu.VMEM((N, D), jnp.bfloat16),
            pltpu.VMEM((N, VP), jnp.bfloat16),
            pltpu.VMEM((D, 1), jnp.float32),
        ],
        interpret=interpret,
    )(x2, lens2, e, lab2, pad2, w_sp, w_enc, b_enc2, w_out)


def kernel(batch_wave, lengths, target_ids, target_labels, target_paddings,
           W_sp, W_enc, b_enc, emb, W_out):
    x2 = batch_wave.reshape(B * TE, SUB * FEAT)
    lens2 = lengths.reshape(B, 1)
    ids = target_ids.reshape(N).astype(jnp.int32)
    lab2 = target_labels.reshape(N, 1).astype(jnp.int32)
    pad2 = target_paddings.reshape(N, 1)
    b_enc2 = b_enc.reshape(1, D)
    e = _sc_gather(ids, emb)
    loss = _tc_main(x2, lens2, e, lab2, pad2, W_sp, W_enc, b_enc2, W_out)
    return loss[0, 0]


# speedup vs baseline: 1.1184x; 1.1184x over previous
import functools

import jax
import jax.numpy as jnp
from jax import lax
from jax.experimental import pallas as pl
from jax.experimental.pallas import tpu as pltpu
from jax.experimental.pallas import tpu_sc as plsc

B, T, FEAT, D, L, V, SUB = 8, 2048, 80, 256, 128, 8000, 4
EPS = 0.1
TE = T // SUB
N = B * L
TV = 1024
NV = (V + TV - 1) // TV


def _make_sc_gather():
    info = plsc.get_sparse_core_info()
    nc, ns = info.num_cores, info.num_subcores
    nw = nc * ns
    b_per_w = N // nw
    mesh = plsc.VectorSubcoreMesh(core_axis_name="c", subcore_axis_name="s")

    @functools.partial(
        pl.kernel,
        mesh=mesh,
        out_type=jax.ShapeDtypeStruct((N, D), jnp.float32),
        scratch_types=[
            pltpu.VMEM((b_per_w,), jnp.int32),
            pltpu.VMEM((b_per_w, D), jnp.float32),
            pltpu.SemaphoreType.DMA,
        ],
    )
    def gather_k(idx_hbm, table_hbm, out_hbm, idx_v, rows_v, sem):
        wid = lax.axis_index("s") * nc + lax.axis_index("c")
        base = wid * b_per_w
        pltpu.sync_copy(idx_hbm.at[pl.ds(base, b_per_w)], idx_v)
        pltpu.async_copy(table_hbm.at[idx_v], rows_v, sem).wait()
        pltpu.sync_copy(rows_v, out_hbm.at[pl.ds(base, b_per_w)])

    return gather_k


_sc_gather_cache = []


def _sc_gather(ids, emb):
    if not _sc_gather_cache:
        _sc_gather_cache.append(_make_sc_gather())
    return _sc_gather_cache[0](ids, emb)


def _tc_body(x_ref, lens_ref, e_ref, lab_ref, pad_ref, wsp_ref, wenc_ref,
             benc_ref, wout_ref, out_ref, dec_ref, m_ref, s_ref, sw_ref,
             zl_ref):
    i = pl.program_id(0)

    @pl.when(i == 0)
    def _encoder():
        h = jnp.dot(x_ref[...], wsp_ref[...].astype(jnp.bfloat16),
                    preferred_element_type=jnp.float32)
        h = jnp.dot(h.astype(jnp.bfloat16), wenc_ref[...].astype(jnp.bfloat16),
                    preferred_element_type=jnp.float32)
        h = jnp.maximum(h + benc_ref[...], 0.0)
        enc_len = lens_ref[...] // SUB
        tmask = (lax.broadcasted_iota(jnp.int32, (B, TE), 1)
                 < enc_len).astype(jnp.float32)
        h = h.reshape(B, TE, D) * tmask[:, :, None]
        ctx = h.sum(axis=1) / jnp.maximum(enc_len, 1).astype(jnp.float32)
        dec_ref[...] = (e_ref[...].reshape(B, L, D)
                        + ctx[:, None, :]).reshape(N, D).astype(jnp.bfloat16)
        m_ref[...] = jnp.full((N, 1), -1e30, jnp.float32)
        s_ref[...] = jnp.zeros((N, 1), jnp.float32)
        sw_ref[...] = jnp.zeros((D, 1), jnp.float32)
        zl_ref[...] = jnp.zeros((N, 1), jnp.float32)

    z = jnp.dot(dec_ref[...], wout_ref[...], preferred_element_type=jnp.float32)
    cols = i * TV + lax.broadcasted_iota(jnp.int32, (1, TV), 1)
    wt = wout_ref[...]

    def _tail(z, wt):
        valid = cols < V
        return (jnp.where(valid, z, -1e30),
                jnp.where(valid, wt, jnp.bfloat16(0.0)))

    zv, wt = lax.cond(i == NV - 1, _tail, lambda z, wt: (z, wt), z, wt)
    sw_ref[...] += jnp.sum(wt, axis=1, keepdims=True, dtype=jnp.float32)
    m_old = m_ref[...]
    m_new = jnp.maximum(m_old, jnp.max(zv, axis=1, keepdims=True))
    s_ref[...] = (s_ref[...] * jnp.exp(m_old - m_new)
                  + jnp.sum(jnp.exp(zv - m_new), axis=1, keepdims=True))
    m_ref[...] = m_new
    zl_ref[...] += jnp.sum(jnp.where(cols == lab_ref[...], z, 0.0),
                           axis=1, keepdims=True)

    @pl.when(i == NV - 1)
    def _finalize():
        lse = m_ref[...] + jnp.log(s_ref[...])
        sz = jnp.dot(dec_ref[...].astype(jnp.float32), sw_ref[...],
                     preferred_element_type=jnp.float32)
        loss_tok = lse - (1.0 - EPS) * zl_ref[...] - EPS * (sz / V)
        tmask = 1.0 - pad_ref[...]
        loss = jnp.sum(loss_tok * tmask) / jnp.maximum(jnp.sum(tmask), 1.0)
        out_ref[...] = loss.reshape(1, 1)


def _tc_main(x2, lens2, e, lab2, pad2, w_sp, w_enc, b_enc2, w_out,
             interpret=False):
    return pl.pallas_call(
        _tc_body,
        grid=(NV,),
        in_specs=[
            pl.BlockSpec((B * TE, SUB * FEAT), lambda i: (0, 0)),
            pl.BlockSpec((B, 1), lambda i: (0, 0)),
            pl.BlockSpec((N, D), lambda i: (0, 0)),
            pl.BlockSpec((N, 1), lambda i: (0, 0)),
            pl.BlockSpec((N, 1), lambda i: (0, 0)),
            pl.BlockSpec((SUB * FEAT, D), lambda i: (0, 0)),
            pl.BlockSpec((D, D), lambda i: (0, 0)),
            pl.BlockSpec((1, D), lambda i: (0, 0)),
            pl.BlockSpec((D, TV), lambda i: (0, i)),
        ],
        out_specs=pl.BlockSpec((1, 1), lambda i: (0, 0)),
        out_shape=jax.ShapeDtypeStruct((1, 1), jnp.float32),
        scratch_shapes=[
            pltpu.VMEM((N, D), jnp.bfloat16),
            pltpu.VMEM((N, 1), jnp.float32),
            pltpu.VMEM((N, 1), jnp.float32),
            pltpu.VMEM((D, 1), jnp.float32),
            pltpu.VMEM((N, 1), jnp.float32),
        ],
        interpret=interpret,
    )(x2, lens2, e, lab2, pad2, w_sp, w_enc, b_enc2, w_out)


def kernel(batch_wave, lengths, target_ids, target_labels, target_paddings,
           W_sp, W_enc, b_enc, emb, W_out):
    x2 = batch_wave.astype(jnp.bfloat16).reshape(B * TE, SUB * FEAT)
    lens2 = lengths.reshape(B, 1)
    ids = target_ids.reshape(N).astype(jnp.int32)
    lab2 = target_labels.reshape(N, 1).astype(jnp.int32)
    pad2 = target_paddings.reshape(N, 1)
    b_enc2 = b_enc.reshape(1, D)
    e = _sc_gather(ids, emb)
    loss = _tc_main(x2, lens2, e, lab2, pad2, W_sp, W_enc, b_enc2,
                    W_out.astype(jnp.bfloat16))
    return loss[0, 0]
